# baseline (speedup 1.0000x reference)
import functools
import math

import jax
import jax.numpy as jnp
from jax import lax
from jax.experimental import pallas as pl
from jax.experimental.pallas import tpu as pltpu
from jax.experimental.pallas import tpu_sc as plsc

B, T = 2, 2048
D = 1024
CD = 256
E = 8
HID2 = 2 * D
SLICE = 1024 // E
N = B * T

TT = 256
NTILES = 2 * N // TT + E
PAD = NTILES * TT

NW = 32
TPW = N // NW
SUB = 64


def _router_body(ce_ref, anchor_ref, g_ref, fi_ref, temp_ref,
                 meta_ref, w2_ref, te_ref, imp_ref, aux_ref):
    ce = ce_ref[...]
    anchor = anchor_ref[...]
    an = anchor / jnp.maximum(
        jnp.sqrt(jnp.sum(anchor * anchor, axis=1, keepdims=True)), 1e-12)
    cn = ce / jnp.maximum(
        jnp.sqrt(jnp.sum(ce * ce, axis=1, keepdims=True)), 1e-12)
    logits = jnp.dot(cn, an.T, preferred_element_type=jnp.float32) * 0.125
    z = (logits + g_ref[...]) / 0.1
    z = z - jnp.max(z, axis=1, keepdims=True)
    ez = jnp.exp(z)
    y = ez / jnp.sum(ez, axis=1, keepdims=True)
    eidx = lax.broadcasted_iota(jnp.int32, (N, E), 1)
    m1 = jnp.max(y, axis=1, keepdims=True)
    i1 = jnp.min(jnp.where(y == m1, eidx, E), axis=1, keepdims=True)
    oh1 = (eidx == i1).astype(jnp.float32)
    ymask = jnp.where(eidx == i1, -jnp.inf, y)
    m2 = jnp.max(ymask, axis=1, keepdims=True)
    i2 = jnp.min(jnp.where(ymask == m2, eidx, E), axis=1, keepdims=True)
    oh2 = (eidx == i2).astype(jnp.float32)
    ew = y * (oh1 + oh2)
    counts = ew[:T, :] + ew[T:, :]
    nelem = T * E
    mean = jnp.sum(counts) / nelem
    var = jnp.sum((counts - mean) ** 2) / (nelem - 1)
    std = jnp.sqrt(var)
    load = counts / (jnp.sum(counts) + 1e-8)
    load_ent = -jnp.sum(load * jnp.log(load + 1e-8))
    routing_loss = 0.5 * (std + load_ent)
    tclip = jnp.clip(temp_ref[0, 0], 0.1, 5.0)
    fi = fi_ref[...] / tclip
    fi = fi - jnp.max(fi, axis=1, keepdims=True)
    efi = jnp.exp(fi)
    imp = efi / jnp.sum(efi, axis=1, keepdims=True)
    imp_ref[...] = imp
    ent = -jnp.sum(imp * jnp.log(imp + 1e-8)) / E
    aux_ref[...] = jnp.broadcast_to(routing_loss - 0.01 * ent, (1, 1))

    def _cumsum0(x):
        k = 1
        while k < N:
            x = x + jnp.concatenate(
                [jnp.zeros((k, E), jnp.float32), x[:-k, :]], axis=0)
            k *= 2
        return x

    c1 = _cumsum0(oh1)
    c2 = _cumsum0(oh2)
    c1tot = c1[N - 1:N, :]
    ctot = c1tot + c2[N - 1:N, :]
    aligned = jnp.floor((ctot + (TT - 1)) * (1.0 / TT)).astype(jnp.float32)
    aligned = aligned * TT
    acc = aligned
    for k in (1, 2, 4):
        acc = acc + jnp.concatenate(
            [jnp.zeros((1, k), jnp.float32), acc[:, :-k]], axis=1)
    start = acc - aligned
    rank0 = jnp.sum((c1 - oh1) * oh1, axis=1, keepdims=True)
    rank1 = jnp.sum(((c2 - oh2) + c1tot) * oh2, axis=1, keepdims=True)
    pos0 = jnp.sum(start * oh1, axis=1, keepdims=True) + rank0
    pos1 = jnp.sum(start * oh2, axis=1, keepdims=True) + rank1
    tvec = lax.broadcasted_iota(jnp.int32, (N, 1), 0)
    q0 = tvec * E + i1
    q1 = tvec * E + i2
    meta_ref[...] = jnp.concatenate(
        [pos0.astype(jnp.int32), pos1.astype(jnp.int32), q0, q1], axis=1)
    w2_ref[0] = jnp.broadcast_to(m1, (N, 128))
    w2_ref[1] = jnp.broadcast_to(m2, (N, 128))
    jstart = (lax.broadcasted_iota(jnp.int32, (1, 128), 1)
              * TT).astype(jnp.float32)
    sb = jnp.broadcast_to(start.reshape(E, 1), (E, 128))
    te = jnp.sum((sb <= jstart).astype(jnp.float32), axis=0, keepdims=True)
    te_ref[...] = (te - 1.0).astype(jnp.int32)


def _dispatch_body(h_hbm, ce_hbm, posT_hbm, wT_hbm,
                   hs_hbm, cs_hbm, ws_hbm,
                   hbuf, cbuf, wsbuf0, wsbuf1, idx0, idx1, sem, sem2):
    wid = lax.axis_index("s") * 2 + lax.axis_index("c")
    tok0 = wid * TPW
    for j in range(TPW // SUB):
        tb = tok0 + j * SUB
        incopies = [
            pltpu.async_copy(h_hbm.at[pl.ds(tb, SUB)], hbuf, sem),
            pltpu.async_copy(ce_hbm.at[pl.ds(tb, SUB)], cbuf, sem),
            pltpu.async_copy(posT_hbm.at[0, pl.ds(tb, SUB)], idx0, sem),
            pltpu.async_copy(posT_hbm.at[1, pl.ds(tb, SUB)], idx1, sem),
            pltpu.async_copy(wT_hbm.at[0, pl.ds(tb, SUB)], wsbuf0, sem),
            pltpu.async_copy(wT_hbm.at[1, pl.ds(tb, SUB)], wsbuf1, sem),
        ]
        for c in incopies:
            c.wait()
        scatters = [
            pltpu.async_copy(wsbuf0, ws_hbm.at[idx0], sem2),
            pltpu.async_copy(hbuf, hs_hbm.at[idx0], sem2),
            pltpu.async_copy(cbuf, cs_hbm.at[idx0], sem2),
            pltpu.async_copy(wsbuf1, ws_hbm.at[idx1], sem2),
            pltpu.async_copy(hbuf, hs_hbm.at[idx1], sem2),
            pltpu.async_copy(cbuf, cs_hbm.at[idx1], sem2),
        ]
        for c in scatters:
            c.wait()


def _mlp_body(te_ref, imp_ref, hs_ref, cs_ref, w1_ref, b1_ref, w2_ref,
              b2_ref, ws_ref, out_ref):
    xh = hs_ref[...] * imp_ref[0]
    a = jnp.dot(xh, w1_ref[0, :D, :], preferred_element_type=jnp.float32)
    a += jnp.dot(cs_ref[...], w1_ref[0, D:, :],
                 preferred_element_type=jnp.float32)
    a += b1_ref[0]
    hdn = 0.5 * a * (1.0 + lax.erf(a * (1.0 / math.sqrt(2.0))))
    out = jnp.dot(hdn, w2_ref[0], preferred_element_type=jnp.float32)
    out += b2_ref[0]
    out_ref[...] = out * ws_ref[:, 0:1]


def _combine_body(outs_hbm, posT_hbm, qT_hbm, z_hbm, outv_hbm,
                  stag, zbuf, gidx, sidx, sem, zsem):
    wid = lax.axis_index("s") * 2 + lax.axis_index("c")
    tok0 = wid * TPW
    row0 = wid * (TPW * E)
    pltpu.sync_copy(z_hbm, zbuf)
    zcopies = [
        pltpu.async_copy(zbuf, outv_hbm.at[pl.ds(row0 + r * 128, 128)], zsem)
        for r in range(TPW * E // 128)
    ]
    for c in zcopies:
        c.wait()
    for j in range(TPW // SUB):
        tb = tok0 + j * SUB
        pltpu.sync_copy(posT_hbm.at[0, pl.ds(tb, SUB)],
                        gidx.at[j, pl.ds(0, SUB)])
        pltpu.sync_copy(posT_hbm.at[1, pl.ds(tb, SUB)],
                        gidx.at[j, pl.ds(SUB, SUB)])
        pltpu.sync_copy(qT_hbm.at[0, pl.ds(tb, SUB)],
                        sidx.at[j, pl.ds(0, SUB)])
        pltpu.sync_copy(qT_hbm.at[1, pl.ds(tb, SUB)],
                        sidx.at[j, pl.ds(SUB, SUB)])
    for j in range(TPW // SUB):
        pltpu.async_copy(outs_hbm.at[gidx.at[j]], stag, sem).wait()
        pltpu.async_copy(stag, outv_hbm.at[sidx.at[j]], sem).wait()


def _dispatch(h2, ce2, posT, wT):
    mesh = plsc.VectorSubcoreMesh(core_axis_name="c", subcore_axis_name="s")
    call = pl.kernel(
        _dispatch_body, mesh=mesh,
        out_type=[
            jax.ShapeDtypeStruct((PAD, D), jnp.float32),
            jax.ShapeDtypeStruct((PAD, CD), jnp.float32),
            jax.ShapeDtypeStruct((PAD, 128), jnp.float32),
        ],
        scratch_types=[
            pltpu.VMEM((SUB, D), jnp.float32),
            pltpu.VMEM((SUB, CD), jnp.float32),
            pltpu.VMEM((SUB, 128), jnp.float32),
            pltpu.VMEM((SUB, 128), jnp.float32),
            pltpu.VMEM((SUB,), jnp.int32),
            pltpu.VMEM((SUB,), jnp.int32),
            pltpu.SemaphoreType.DMA,
            pltpu.SemaphoreType.DMA,
        ],
    )
    return call(h2, ce2, posT, wT)


def _combine(out_sorted, posT, qT, zrows):
    mesh = plsc.VectorSubcoreMesh(core_axis_name="c", subcore_axis_name="s")
    call = pl.kernel(
        _combine_body, mesh=mesh,
        out_type=jax.ShapeDtypeStruct((N * E, SLICE), jnp.float32),
        scratch_types=[
            pltpu.VMEM((2 * SUB, SLICE), jnp.float32),
            pltpu.VMEM((128, SLICE), jnp.float32),
            pltpu.VMEM((TPW // SUB, 2 * SUB), jnp.int32),
            pltpu.VMEM((TPW // SUB, 2 * SUB), jnp.int32),
            pltpu.SemaphoreType.DMA,
            pltpu.SemaphoreType.DMA,
        ],
    )
    return call(out_sorted, posT, qT, zrows)


def kernel(h, code_emb, code_anchor, feature_importance,
           importance_temperature, W1, b1, W2, b2):
    h2 = h.reshape(N, D)
    ce2 = code_emb.reshape(N, CD)
    g = jax.random.gumbel(jax.random.key(42), (N, E), dtype=jnp.float32)
    temp = importance_temperature.reshape(1, 1)

    meta, w2pair, te, imp, aux = pl.pallas_call(
        _router_body,
        out_shape=[
            jax.ShapeDtypeStruct((N, 4), jnp.int32),
            jax.ShapeDtypeStruct((2, N, 128), jnp.float32),
            jax.ShapeDtypeStruct((1, 128), jnp.int32),
            jax.ShapeDtypeStruct((E, D), jnp.float32),
            jax.ShapeDtypeStruct((1, 1), jnp.float32),
        ],
    )(ce2, code_anchor, g, feature_importance, temp)

    posT = meta[:, 0:2].T
    qT = meta[:, 2:4].T
    wT = w2pair

    hs, cs, ws = _dispatch(h2, ce2, posT, wT)

    out_sorted = pl.pallas_call(
        _mlp_body,
        grid_spec=pltpu.PrefetchScalarGridSpec(
            num_scalar_prefetch=1,
            grid=(NTILES,),
            in_specs=[
                pl.BlockSpec((1, 1, D), lambda i, te_s: (te_s[i], 0, 0)),
                pl.BlockSpec((TT, D), lambda i, te_s: (i, 0)),
                pl.BlockSpec((TT, CD), lambda i, te_s: (i, 0)),
                pl.BlockSpec((1, D + CD, HID2),
                             lambda i, te_s: (te_s[i], 0, 0)),
                pl.BlockSpec((1, 1, HID2), lambda i, te_s: (te_s[i], 0, 0)),
                pl.BlockSpec((1, HID2, SLICE),
                             lambda i, te_s: (te_s[i], 0, 0)),
                pl.BlockSpec((1, 1, SLICE), lambda i, te_s: (te_s[i], 0, 0)),
                pl.BlockSpec((TT, 128), lambda i, te_s: (i, 0)),
            ],
            out_specs=pl.BlockSpec((TT, SLICE), lambda i, te_s: (i, 0)),
        ),
        out_shape=jax.ShapeDtypeStruct((PAD, SLICE), jnp.float32),
    )(te.reshape(128), imp.reshape(E, 1, D), hs, cs, W1,
      b1.reshape(E, 1, HID2), W2, b2.reshape(E, 1, SLICE), ws)

    zrows = jnp.zeros((128, SLICE), jnp.float32)
    outv = _combine(out_sorted, posT, qT, zrows)

    return outv.reshape(B, T, E * SLICE), aux.reshape(())


# speedup vs baseline: 1.0259x; 1.0259x over previous
import functools
import math

import jax
import jax.numpy as jnp
from jax import lax
from jax.experimental import pallas as pl
from jax.experimental.pallas import tpu as pltpu
from jax.experimental.pallas import tpu_sc as plsc

B, T = 2, 2048
D = 1024
CD = 256
E = 8
HID2 = 2 * D
SLICE = 1024 // E
N = B * T

TT = 512
NTILES = 2 * N // TT + E
PAD = NTILES * TT

NW = 32
TPW = N // NW
SUB = 64


def _router_body(ce_ref, anchor_ref, g_ref, fi_ref, temp_ref,
                 meta_ref, w2_ref, te_ref, imp_ref, aux_ref):
    ce = ce_ref[...]
    anchor = anchor_ref[...]
    an = anchor / jnp.maximum(
        jnp.sqrt(jnp.sum(anchor * anchor, axis=1, keepdims=True)), 1e-12)
    cn = ce / jnp.maximum(
        jnp.sqrt(jnp.sum(ce * ce, axis=1, keepdims=True)), 1e-12)
    logits = jnp.dot(cn, an.T, preferred_element_type=jnp.float32) * 0.125
    z = (logits + g_ref[...]) / 0.1
    z = z - jnp.max(z, axis=1, keepdims=True)
    ez = jnp.exp(z)
    y = ez / jnp.sum(ez, axis=1, keepdims=True)
    eidx = lax.broadcasted_iota(jnp.int32, (N, E), 1)
    m1 = jnp.max(y, axis=1, keepdims=True)
    i1 = jnp.min(jnp.where(y == m1, eidx, E), axis=1, keepdims=True)
    oh1 = (eidx == i1).astype(jnp.float32)
    ymask = jnp.where(eidx == i1, -jnp.inf, y)
    m2 = jnp.max(ymask, axis=1, keepdims=True)
    i2 = jnp.min(jnp.where(ymask == m2, eidx, E), axis=1, keepdims=True)
    oh2 = (eidx == i2).astype(jnp.float32)
    ew = y * (oh1 + oh2)
    counts = ew[:T, :] + ew[T:, :]
    nelem = T * E
    mean = jnp.sum(counts) / nelem
    var = jnp.sum((counts - mean) ** 2) / (nelem - 1)
    std = jnp.sqrt(var)
    load = counts / (jnp.sum(counts) + 1e-8)
    load_ent = -jnp.sum(load * jnp.log(load + 1e-8))
    routing_loss = 0.5 * (std + load_ent)
    tclip = jnp.clip(temp_ref[0, 0], 0.1, 5.0)
    fi = fi_ref[...] / tclip
    fi = fi - jnp.max(fi, axis=1, keepdims=True)
    efi = jnp.exp(fi)
    imp = efi / jnp.sum(efi, axis=1, keepdims=True)
    imp_ref[...] = imp
    ent = -jnp.sum(imp * jnp.log(imp + 1e-8)) / E
    aux_ref[...] = jnp.broadcast_to(routing_loss - 0.01 * ent, (1, 1))

    def _cumsum0(x):
        k = 1
        while k < N:
            x = x + jnp.concatenate(
                [jnp.zeros((k, E), jnp.float32), x[:-k, :]], axis=0)
            k *= 2
        return x

    c1 = _cumsum0(oh1)
    c2 = _cumsum0(oh2)
    c1tot = c1[N - 1:N, :]
    ctot = c1tot + c2[N - 1:N, :]
    aligned = jnp.floor((ctot + (TT - 1)) * (1.0 / TT)).astype(jnp.float32)
    aligned = aligned * TT
    acc = aligned
    for k in (1, 2, 4):
        acc = acc + jnp.concatenate(
            [jnp.zeros((1, k), jnp.float32), acc[:, :-k]], axis=1)
    start = acc - aligned
    rank0 = jnp.sum((c1 - oh1) * oh1, axis=1, keepdims=True)
    rank1 = jnp.sum(((c2 - oh2) + c1tot) * oh2, axis=1, keepdims=True)
    pos0 = jnp.sum(start * oh1, axis=1, keepdims=True) + rank0
    pos1 = jnp.sum(start * oh2, axis=1, keepdims=True) + rank1
    tvec = lax.broadcasted_iota(jnp.int32, (N, 1), 0)
    q0 = tvec * E + i1
    q1 = tvec * E + i2
    meta_ref[...] = jnp.concatenate(
        [pos0.astype(jnp.int32), pos1.astype(jnp.int32), q0, q1], axis=1)
    w2_ref[0] = jnp.broadcast_to(m1, (N, 128))
    w2_ref[1] = jnp.broadcast_to(m2, (N, 128))
    jstart = (lax.broadcasted_iota(jnp.int32, (1, 128), 1)
              * TT).astype(jnp.float32)
    sb = jnp.broadcast_to(start.reshape(E, 1), (E, 128))
    te = jnp.sum((sb <= jstart).astype(jnp.float32), axis=0, keepdims=True)
    te_ref[...] = (te - 1.0).astype(jnp.int32)


def _dispatch_body(h_hbm, ce_hbm, posT_hbm, wT_hbm,
                   hs_hbm, cs_hbm, ws_hbm,
                   hbuf, cbuf, wsbuf0, wsbuf1, idx0, idx1, sem, sem2):
    wid = lax.axis_index("s") * 2 + lax.axis_index("c")
    tok0 = wid * TPW
    for j in range(TPW // SUB):
        tb = tok0 + j * SUB
        incopies = [
            pltpu.async_copy(h_hbm.at[pl.ds(tb, SUB)], hbuf, sem),
            pltpu.async_copy(ce_hbm.at[pl.ds(tb, SUB)], cbuf, sem),
            pltpu.async_copy(posT_hbm.at[0, pl.ds(tb, SUB)], idx0, sem),
            pltpu.async_copy(posT_hbm.at[1, pl.ds(tb, SUB)], idx1, sem),
            pltpu.async_copy(wT_hbm.at[0, pl.ds(tb, SUB)], wsbuf0, sem),
            pltpu.async_copy(wT_hbm.at[1, pl.ds(tb, SUB)], wsbuf1, sem),
        ]
        for c in incopies:
            c.wait()
        scatters = [
            pltpu.async_copy(wsbuf0, ws_hbm.at[idx0], sem2),
            pltpu.async_copy(hbuf, hs_hbm.at[idx0], sem2),
            pltpu.async_copy(cbuf, cs_hbm.at[idx0], sem2),
            pltpu.async_copy(wsbuf1, ws_hbm.at[idx1], sem2),
            pltpu.async_copy(hbuf, hs_hbm.at[idx1], sem2),
            pltpu.async_copy(cbuf, cs_hbm.at[idx1], sem2),
        ]
        for c in scatters:
            c.wait()


def _mlp_body(te_ref, imp_ref, hs_ref, cs_ref, w1_ref, b1_ref, w2_ref,
              b2_ref, ws_ref, out_ref):
    xh = hs_ref[...] * imp_ref[0]
    a = jnp.dot(xh, w1_ref[0, :D, :], preferred_element_type=jnp.float32)
    a += jnp.dot(cs_ref[...], w1_ref[0, D:, :],
                 preferred_element_type=jnp.float32)
    a += b1_ref[0]
    hdn = 0.5 * a * (1.0 + lax.erf(a * (1.0 / math.sqrt(2.0))))
    out = jnp.dot(hdn, w2_ref[0], preferred_element_type=jnp.float32)
    out += b2_ref[0]
    out_ref[...] = out * ws_ref[:, 0:1]


def _combine_body(outs_hbm, posT_hbm, qT_hbm, z_hbm, outv_hbm,
                  stag, zbuf, gidx, sidx, sem, zsem):
    wid = lax.axis_index("s") * 2 + lax.axis_index("c")
    tok0 = wid * TPW
    row0 = wid * (TPW * E)
    pltpu.sync_copy(z_hbm, zbuf)
    zcopies = [
        pltpu.async_copy(zbuf, outv_hbm.at[pl.ds(row0 + r * 128, 128)], zsem)
        for r in range(TPW * E // 128)
    ]
    idxcopies = []
    for j in range(TPW // SUB):
        tb = tok0 + j * SUB
        idxcopies += [
            pltpu.async_copy(posT_hbm.at[0, pl.ds(tb, SUB)],
                             gidx.at[j, pl.ds(0, SUB)], sem),
            pltpu.async_copy(posT_hbm.at[1, pl.ds(tb, SUB)],
                             gidx.at[j, pl.ds(SUB, SUB)], sem),
            pltpu.async_copy(qT_hbm.at[0, pl.ds(tb, SUB)],
                             sidx.at[j, pl.ds(0, SUB)], sem),
            pltpu.async_copy(qT_hbm.at[1, pl.ds(tb, SUB)],
                             sidx.at[j, pl.ds(SUB, SUB)], sem),
        ]
    for c in idxcopies:
        c.wait()
    for c in zcopies:
        c.wait()
    for j in range(TPW // SUB):
        pltpu.async_copy(outs_hbm.at[gidx.at[j]], stag, sem).wait()
        pltpu.async_copy(stag, outv_hbm.at[sidx.at[j]], sem).wait()


def _dispatch(h2, ce2, posT, wT):
    mesh = plsc.VectorSubcoreMesh(core_axis_name="c", subcore_axis_name="s")
    call = pl.kernel(
        _dispatch_body, mesh=mesh,
        out_type=[
            jax.ShapeDtypeStruct((PAD, D), jnp.float32),
            jax.ShapeDtypeStruct((PAD, CD), jnp.float32),
            jax.ShapeDtypeStruct((PAD, 128), jnp.float32),
        ],
        scratch_types=[
            pltpu.VMEM((SUB, D), jnp.float32),
            pltpu.VMEM((SUB, CD), jnp.float32),
            pltpu.VMEM((SUB, 128), jnp.float32),
            pltpu.VMEM((SUB, 128), jnp.float32),
            pltpu.VMEM((SUB,), jnp.int32),
            pltpu.VMEM((SUB,), jnp.int32),
            pltpu.SemaphoreType.DMA,
            pltpu.SemaphoreType.DMA,
        ],
    )
    return call(h2, ce2, posT, wT)


def _combine(out_sorted, posT, qT, zrows):
    mesh = plsc.VectorSubcoreMesh(core_axis_name="c", subcore_axis_name="s")
    call = pl.kernel(
        _combine_body, mesh=mesh,
        out_type=jax.ShapeDtypeStruct((N * E, SLICE), jnp.float32),
        scratch_types=[
            pltpu.VMEM((2 * SUB, SLICE), jnp.float32),
            pltpu.VMEM((128, SLICE), jnp.float32),
            pltpu.VMEM((TPW // SUB, 2 * SUB), jnp.int32),
            pltpu.VMEM((TPW // SUB, 2 * SUB), jnp.int32),
            pltpu.SemaphoreType.DMA,
            pltpu.SemaphoreType.DMA,
        ],
    )
    return call(out_sorted, posT, qT, zrows)


def kernel(h, code_emb, code_anchor, feature_importance,
           importance_temperature, W1, b1, W2, b2):
    h2 = h.reshape(N, D)
    ce2 = code_emb.reshape(N, CD)
    g = jax.random.gumbel(jax.random.key(42), (N, E), dtype=jnp.float32)
    temp = importance_temperature.reshape(1, 1)

    meta, w2pair, te, imp, aux = pl.pallas_call(
        _router_body,
        out_shape=[
            jax.ShapeDtypeStruct((N, 4), jnp.int32),
            jax.ShapeDtypeStruct((2, N, 128), jnp.float32),
            jax.ShapeDtypeStruct((1, 128), jnp.int32),
            jax.ShapeDtypeStruct((E, D), jnp.float32),
            jax.ShapeDtypeStruct((1, 1), jnp.float32),
        ],
    )(ce2, code_anchor, g, feature_importance, temp)

    posT = meta[:, 0:2].T
    qT = meta[:, 2:4].T
    wT = w2pair

    hs, cs, ws = _dispatch(h2, ce2, posT, wT)

    out_sorted = pl.pallas_call(
        _mlp_body,
        grid_spec=pltpu.PrefetchScalarGridSpec(
            num_scalar_prefetch=1,
            grid=(NTILES,),
            in_specs=[
                pl.BlockSpec((1, 1, D), lambda i, te_s: (te_s[i], 0, 0)),
                pl.BlockSpec((TT, D), lambda i, te_s: (i, 0)),
                pl.BlockSpec((TT, CD), lambda i, te_s: (i, 0)),
                pl.BlockSpec((1, D + CD, HID2),
                             lambda i, te_s: (te_s[i], 0, 0)),
                pl.BlockSpec((1, 1, HID2), lambda i, te_s: (te_s[i], 0, 0)),
                pl.BlockSpec((1, HID2, SLICE),
                             lambda i, te_s: (te_s[i], 0, 0)),
                pl.BlockSpec((1, 1, SLICE), lambda i, te_s: (te_s[i], 0, 0)),
                pl.BlockSpec((TT, 128), lambda i, te_s: (i, 0)),
            ],
            out_specs=pl.BlockSpec((TT, SLICE), lambda i, te_s: (i, 0)),
        ),
        out_shape=jax.ShapeDtypeStruct((PAD, SLICE), jnp.float32),
    )(te.reshape(128), imp.reshape(E, 1, D), hs, cs, W1,
      b1.reshape(E, 1, HID2), W2, b2.reshape(E, 1, SLICE), ws)

    zrows = jnp.zeros((128, SLICE), jnp.float32)
    outv = _combine(out_sorted, posT, qT, zrows)

    return outv.reshape(B, T, E * SLICE), aux.reshape(())
